# baseline (speedup 1.0000x reference)
import functools

import jax
import jax.numpy as jnp
from jax import lax
from jax.experimental import pallas as pl
from jax.experimental.pallas import tpu as pltpu
from jax.experimental.pallas import tpu_sc as plsc

_NC, _NS = 2, 16
_NW = _NC * _NS
_CH = 4
_NBUF = 4
_L = 16
_UNROLL = 8
_T_SC = 4096
_BT = 2048


def _sc_add(x, pos_table):
    B, T, D = x.shape
    rows_w = T // _NW
    nch = rows_w // _CH
    ncyc = nch // _NBUF

    mesh = plsc.VectorSubcoreMesh(core_axis_name="c", subcore_axis_name="s")

    scratch = (
        [pltpu.VMEM((_CH, D), jnp.float32) for _ in range(_NBUF)]
        + [pltpu.VMEM((B, _CH, D), jnp.float32) for _ in range(_NBUF)]
        + [pltpu.SemaphoreType.DMA for _ in range(2 * _NBUF)]
    )

    @functools.partial(
        pl.kernel,
        out_type=jax.ShapeDtypeStruct((B, T, D), x.dtype),
        mesh=mesh,
        scratch_types=scratch,
    )
    def run(x_hbm, p_hbm, o_hbm, *scr):
        pbuf = scr[:_NBUF]
        xbuf = scr[_NBUF : 2 * _NBUF]
        sin = scr[2 * _NBUF : 3 * _NBUF]
        sout = scr[3 * _NBUF :]

        wid = lax.axis_index("s") * _NC + lax.axis_index("c")
        base = wid * rows_w

        def issue_in(cg, k):
            t0 = base + cg * _CH
            pltpu.async_copy(p_hbm.at[pl.ds(t0, _CH)], pbuf[k], sin[k])
            for b in range(B):
                pltpu.async_copy(x_hbm.at[b, pl.ds(t0, _CH)], xbuf[k].at[b], sin[k])

        def wait_in(k):
            pltpu.make_async_copy(p_hbm.at[pl.ds(0, _CH)], pbuf[k], sin[k]).wait()
            pltpu.make_async_copy(
                x_hbm.at[pl.ds(0, B), pl.ds(0, _CH)], xbuf[k], sin[k]
            ).wait()

        def issue_out(cg, k):
            t0 = base + cg * _CH
            for b in range(B):
                pltpu.async_copy(xbuf[k].at[b], o_hbm.at[b, pl.ds(t0, _CH)], sout[k])

        def wait_out(k):
            pltpu.make_async_copy(
                xbuf[k], o_hbm.at[pl.ds(0, B), pl.ds(0, _CH)], sout[k]
            ).wait()

        def compute(k):
            for r in range(_CH):

                @plsc.parallel_loop(0, D, step=_L, unroll=_UNROLL)
                def _(j, r=r, k=k):
                    sl = pl.ds(j, _L)
                    pv = pbuf[k][r, sl]
                    for b in range(B):
                        plsc.addupdate(xbuf[k].at[b, r, sl], pv)

        issue_in(0, 0)
        issue_in(1, 1)

        def cycle(g, carry):
            for k in range(_NBUF):
                cg = g * _NBUF + k
                wait_in(k)
                compute(k)
                issue_out(cg, k)
                kp = (k + 2) % _NBUF

                @pl.when(cg >= 2)
                def _(kp=kp):
                    wait_out(kp)

                @pl.when(cg < nch - 2)
                def _(cg=cg, kp=kp):
                    issue_in(cg + 2, kp)

            return carry

        lax.fori_loop(0, ncyc, cycle, 0)
        wait_out((nch - 2) % _NBUF)
        wait_out((nch - 1) % _NBUF)

    return run(x, pos_table)


def _tc_body(x_ref, p_ref, o_ref):
    o_ref[...] = x_ref[...] + p_ref[...]


def _tc_add(x, pos_table):
    B, T, D = x.shape
    bt = min(_BT, T)
    return pl.pallas_call(
        _tc_body,
        grid=(T // bt, B),
        in_specs=[
            pl.BlockSpec((1, bt, D), lambda t, b: (b, t, 0)),
            pl.BlockSpec((bt, D), lambda t, b: (t, 0)),
        ],
        out_specs=pl.BlockSpec((1, bt, D), lambda t, b: (b, t, 0)),
        out_shape=jax.ShapeDtypeStruct(x.shape, x.dtype),
    )(x, pos_table)


def kernel(x, pos_table):
    B, T, D = x.shape
    t_sc = _T_SC if 0 < _T_SC < T and _T_SC % 512 == 0 else 0
    if t_sc == 0 or (T - t_sc) % _BT != 0:
        return _tc_add(x, pos_table)
    t_tc = T - t_sc
    out_tc = _tc_add(x[:, :t_tc], pos_table[:t_tc])
    out_sc = _sc_add(x[:, t_tc:], pos_table[t_tc:])
    return jnp.concatenate([out_tc, out_sc], axis=1)


# speedup vs baseline: 1.0553x; 1.0553x over previous
import functools

import jax
import jax.numpy as jnp
from jax import lax
from jax.experimental import pallas as pl
from jax.experimental.pallas import tpu as pltpu
from jax.experimental.pallas import tpu_sc as plsc

_NC, _NS = 2, 16
_NW = _NC * _NS
_CH = 4
_NBUF = 4
_L = 16
_UNROLL = 8
_T_SC = 4096
_BT = 2048


def _sc_add(x, pos_table):
    B, T, D = x.shape
    rows_w = T // _NW
    nch = rows_w // _CH
    ncyc = nch // _NBUF

    mesh = plsc.VectorSubcoreMesh(core_axis_name="c", subcore_axis_name="s")

    scratch = (
        [pltpu.VMEM((_CH, D), jnp.float32) for _ in range(_NBUF)]
        + [pltpu.VMEM((B, _CH, D), jnp.float32) for _ in range(_NBUF)]
        + [pltpu.SemaphoreType.DMA for _ in range(2 * _NBUF)]
    )

    @functools.partial(
        pl.kernel,
        out_type=jax.ShapeDtypeStruct((B, T, D), x.dtype),
        mesh=mesh,
        scratch_types=scratch,
    )
    def run(x_hbm, p_hbm, o_hbm, *scr):
        pbuf = scr[:_NBUF]
        xbuf = scr[_NBUF : 2 * _NBUF]
        sin = scr[2 * _NBUF : 3 * _NBUF]
        sout = scr[3 * _NBUF :]

        wid = lax.axis_index("s") * _NC + lax.axis_index("c")
        base = wid * rows_w

        def issue_in(cg, k):
            t0 = base + cg * _CH
            pltpu.async_copy(p_hbm.at[pl.ds(t0, _CH)], pbuf[k], sin[k])
            for b in range(B):
                pltpu.async_copy(x_hbm.at[b, pl.ds(t0, _CH)], xbuf[k].at[b], sin[k])

        def wait_in(k):
            pltpu.make_async_copy(p_hbm.at[pl.ds(0, _CH)], pbuf[k], sin[k]).wait()
            pltpu.make_async_copy(
                x_hbm.at[pl.ds(0, B), pl.ds(0, _CH)], xbuf[k], sin[k]
            ).wait()

        def issue_out(cg, k):
            t0 = base + cg * _CH
            for b in range(B):
                pltpu.async_copy(xbuf[k].at[b], o_hbm.at[b, pl.ds(t0, _CH)], sout[k])

        def wait_out(k):
            pltpu.make_async_copy(
                xbuf[k], o_hbm.at[pl.ds(0, B), pl.ds(0, _CH)], sout[k]
            ).wait()

        def compute(k):
            for r in range(_CH):

                @plsc.parallel_loop(0, D, step=_L, unroll=_UNROLL)
                def _(j, r=r, k=k):
                    sl = pl.ds(j, _L)
                    pv = pbuf[k][r, sl]
                    for b in range(B):
                        plsc.addupdate(xbuf[k].at[b, r, sl], pv)

        issue_in(0, 0)
        issue_in(1, 1)

        def cycle(g, carry):
            for k in range(_NBUF):
                cg = g * _NBUF + k
                wait_in(k)
                compute(k)
                issue_out(cg, k)
                kp = (k + 2) % _NBUF

                @pl.when(cg >= 2)
                def _(kp=kp):
                    wait_out(kp)

                @pl.when(cg < nch - 2)
                def _(cg=cg, kp=kp):
                    issue_in(cg + 2, kp)

            return carry

        lax.fori_loop(0, ncyc, cycle, 0)
        wait_out((nch - 2) % _NBUF)
        wait_out((nch - 1) % _NBUF)

    return run(x, pos_table)


def _tc_body(x_ref, p_ref, o_ref):
    o_ref[...] = x_ref[...] + p_ref[...]


def _tc_add(x, pos_table):
    B, T, D = x.shape
    bt = min(_BT, T)
    return pl.pallas_call(
        _tc_body,
        grid=(T // bt, B),
        in_specs=[
            pl.BlockSpec((1, bt, D), lambda t, b: (b, t, 0)),
            pl.BlockSpec((bt, D), lambda t, b: (t, 0)),
        ],
        out_specs=pl.BlockSpec((1, bt, D), lambda t, b: (b, t, 0)),
        out_shape=jax.ShapeDtypeStruct(x.shape, x.dtype),
    )(x, pos_table)


_B_SC = 1


def kernel(x, pos_table):
    B, T, D = x.shape
    if not (0 < _B_SC < B) or T % (_NW * _CH * _NBUF) != 0:
        return _tc_add(x, pos_table)
    b_tc = B - _B_SC
    out_tc = _tc_add(x[:b_tc], pos_table)
    out_sc = _sc_add(x[b_tc:], pos_table)
    return jnp.concatenate([out_tc, out_sc], axis=0)


# speedup vs baseline: 2.8399x; 2.6910x over previous
import jax
import jax.numpy as jnp
from jax.experimental import pallas as pl

_BT = 512


def _body(x_ref, p_ref, o_ref):
    o_ref[...] = x_ref[...] + p_ref[...]


def kernel(x, pos_table):
    B, T, D = x.shape
    bt = _BT if T % _BT == 0 else T
    return pl.pallas_call(
        _body,
        grid=(T // bt, B),
        in_specs=[
            pl.BlockSpec((1, bt, D), lambda t, b: (b, t, 0)),
            pl.BlockSpec((bt, D), lambda t, b: (t, 0)),
        ],
        out_specs=pl.BlockSpec((1, bt, D), lambda t, b: (b, t, 0)),
        out_shape=jax.ShapeDtypeStruct(x.shape, x.dtype),
    )(x, pos_table)


# speedup vs baseline: 3.1562x; 1.1114x over previous
import jax
import jax.numpy as jnp
from jax.experimental import pallas as pl

_BT = 1024


def _body(x_ref, p_ref, o_ref):
    o_ref[...] = x_ref[...] + p_ref[...]


def kernel(x, pos_table):
    B, T, D = x.shape
    bt = _BT if T % _BT == 0 else T
    return pl.pallas_call(
        _body,
        grid=(T // bt, B),
        in_specs=[
            pl.BlockSpec((1, bt, D), lambda t, b: (b, t, 0)),
            pl.BlockSpec((bt, D), lambda t, b: (t, 0)),
        ],
        out_specs=pl.BlockSpec((1, bt, D), lambda t, b: (b, t, 0)),
        out_shape=jax.ShapeDtypeStruct(x.shape, x.dtype),
    )(x, pos_table)


# speedup vs baseline: 3.2915x; 1.0429x over previous
import jax
import jax.numpy as jnp
from jax.experimental import pallas as pl

_BT = 2048


def _body(x_ref, p_ref, o_ref):
    o_ref[...] = x_ref[...] + p_ref[...]


def kernel(x, pos_table):
    B, T, D = x.shape
    bt = _BT if T % _BT == 0 else T
    return pl.pallas_call(
        _body,
        grid=(T // bt, B),
        in_specs=[
            pl.BlockSpec((1, bt, D), lambda t, b: (b, t, 0)),
            pl.BlockSpec((bt, D), lambda t, b: (t, 0)),
        ],
        out_specs=pl.BlockSpec((1, bt, D), lambda t, b: (b, t, 0)),
        out_shape=jax.ShapeDtypeStruct(x.shape, x.dtype),
    )(x, pos_table)
